# baseline (speedup 1.0000x reference)
import jax
import jax.numpy as jnp
from jax.experimental import pallas as pl
from jax.experimental.pallas import tpu as pltpu
from jax.experimental.pallas import tpu_sc as plsc

_C = 2
_K = 320
_D = 384
_DIN = 768
_CK = _C * _K
_TM = 1024
_GW = 128
_NCH = 2


def _logits_argmax_body(x_ref, wt0_ref, wt1_ref, b_ref, logits_ref, idx_ref):
    x = x_ref[...].astype(jnp.bfloat16)
    dn = (((1,), (0,)), ((), ()))
    l0 = jax.lax.dot_general(x, wt0_ref[...], dn,
                             preferred_element_type=jnp.float32)
    l1 = jax.lax.dot_general(x, wt1_ref[...], dn,
                             preferred_element_type=jnp.float32)
    l0 = l0 + b_ref[0, 0, :][None, :]
    l1 = l1 + b_ref[0, 1, :][None, :]
    logits_ref[:, 0, :] = l0
    logits_ref[:, 1, :] = l1
    lane = jax.lax.broadcasted_iota(
        jnp.int32, (_TM, _K), 1).astype(jnp.float32)
    big = jnp.float32(_CK)
    m0 = jnp.max(l0, axis=1, keepdims=True)
    m1 = jnp.max(l1, axis=1, keepdims=True)
    i0 = jnp.min(jnp.where(l0 == m0, lane, big),
                 axis=1, keepdims=True).astype(jnp.int32)
    i1 = jnp.min(jnp.where(l1 == m1, lane, big),
                 axis=1, keepdims=True).astype(jnp.int32)
    idx_ref[0:1, :] = i0.T
    idx_ref[1:2, :] = i1.T + _K


def _logits_and_indices(xf, wt0, wt1, b3d, chunk, logits_carry):
    t = xf.shape[0]
    half = t // _NCH
    base = chunk * (half // _TM)
    in_specs = [
        pl.BlockSpec((_TM, _DIN), lambda i: (base + i, 0)),
        pl.BlockSpec((_DIN, _K), lambda i: (0, 0)),
        pl.BlockSpec((_DIN, _K), lambda i: (0, 0)),
        pl.BlockSpec((1, _C, _K), lambda i: (0, 0, 0)),
    ]
    args = [xf, wt0, wt1, b3d]
    aliases = {}
    body = _logits_argmax_body
    if logits_carry is not None:
        in_specs.append(pl.BlockSpec(memory_space=pl.ANY))
        args.append(logits_carry)
        aliases = {4: 0}
        body = lambda x, w0, w1, b, carry, louts, iouts: (
            _logits_argmax_body(x, w0, w1, b, louts, iouts))
    return pl.pallas_call(
        body,
        grid=(half // _TM,),
        in_specs=in_specs,
        out_specs=[
            pl.BlockSpec((_TM, _C, _K), lambda i: (base + i, 0, 0)),
            pl.BlockSpec((_C, _TM), lambda i: (0, i)),
        ],
        out_shape=[
            jax.ShapeDtypeStruct((t, _C, _K), jnp.float32),
            jax.ShapeDtypeStruct((_C, half), jnp.int32),
        ],
        input_output_aliases=aliases,
    )(*args)


def _sc_gather(qc, idx2):
    t = idx2.shape[1]
    mesh = plsc.VectorSubcoreMesh(core_axis_name="core",
                                  subcore_axis_name="subcore")

    @pl.kernel(out_type=jax.ShapeDtypeStruct((t, _C * _D), jnp.float32),
               mesh=mesh)
    def gather_kernel(qc_hbm, i_hbm, o_hbm):
        def body(i_vmem, o_vmem):
            pltpu.sync_copy(qc_hbm.at[i_vmem.at[0]], o_vmem)

        pltpu.emit_pipeline(
            body,
            grid=(t // _GW, _C),
            in_specs=[pl.BlockSpec((1, _GW), lambda i, c: (c, i))],
            out_specs=[pl.BlockSpec((_GW, _D), lambda i, c: (i, c))],
            core_axis_name=("core", "subcore"),
            dimension_semantics=(pltpu.PARALLEL, pltpu.PARALLEL),
        )(i_hbm, o_hbm)

    return gather_kernel(qc, idx2)


def kernel(x, quantization_choices, W, b):
    B, S, _ = x.shape
    t = B * S
    xf = x.reshape(t, _DIN)
    wt = W.T.astype(jnp.bfloat16)
    wt0, wt1, b3d = wt[:, :_K], wt[:, _K:], b.reshape(1, _C, _K)
    logits = None
    qs = []
    for k in range(_NCH):
        logits, idx_k = _logits_and_indices(xf, wt0, wt1, b3d, k, logits)
        qs.append(_sc_gather(quantization_choices, idx_k))
    q = jnp.concatenate(qs, axis=0).reshape(B, S, _C * _D)
    return q, logits.reshape(B, S, _C, _K)


# speedup vs baseline: 1.1599x; 1.1599x over previous
import jax
import jax.numpy as jnp
from jax.experimental import pallas as pl
from jax.experimental.pallas import tpu as pltpu
from jax.experimental.pallas import tpu_sc as plsc

_C = 2
_K = 320
_D = 384
_DIN = 768
_CK = _C * _K
_TM = 1024
_GW = 384
_NCH = 1


def _logits_argmax_body(x_ref, wt0_ref, wt1_ref, b_ref, logits_ref, idx_ref):
    x = x_ref[...].astype(jnp.bfloat16)
    dn = (((1,), (0,)), ((), ()))
    l0 = jax.lax.dot_general(x, wt0_ref[...], dn,
                             preferred_element_type=jnp.float32)
    l1 = jax.lax.dot_general(x, wt1_ref[...], dn,
                             preferred_element_type=jnp.float32)
    l0 = l0 + b_ref[0, 0, :][None, :]
    l1 = l1 + b_ref[0, 1, :][None, :]
    logits_ref[:, 0, :] = l0
    logits_ref[:, 1, :] = l1
    lane = jax.lax.broadcasted_iota(
        jnp.int32, (_TM, _K), 1).astype(jnp.float32)
    big = jnp.float32(_CK)
    m0 = jnp.max(l0, axis=1, keepdims=True)
    m1 = jnp.max(l1, axis=1, keepdims=True)
    i0 = jnp.min(jnp.where(l0 == m0, lane, big),
                 axis=1, keepdims=True).astype(jnp.int32)
    i1 = jnp.min(jnp.where(l1 == m1, lane, big),
                 axis=1, keepdims=True).astype(jnp.int32)
    idx_ref[0:1, :] = i0.T
    idx_ref[1:2, :] = i1.T + _K


def _logits_and_indices(xf, wt0, wt1, b3d, chunk, logits_carry):
    t = xf.shape[0]
    half = t // _NCH
    base = chunk * (half // _TM)
    in_specs = [
        pl.BlockSpec((_TM, _DIN), lambda i: (base + i, 0)),
        pl.BlockSpec((_DIN, _K), lambda i: (0, 0)),
        pl.BlockSpec((_DIN, _K), lambda i: (0, 0)),
        pl.BlockSpec((1, _C, _K), lambda i: (0, 0, 0)),
    ]
    args = [xf, wt0, wt1, b3d]
    aliases = {}
    body = _logits_argmax_body
    if logits_carry is not None:
        in_specs.append(pl.BlockSpec(memory_space=pl.ANY))
        args.append(logits_carry)
        aliases = {4: 0}
        body = lambda x, w0, w1, b, carry, louts, iouts: (
            _logits_argmax_body(x, w0, w1, b, louts, iouts))
    return pl.pallas_call(
        body,
        grid=(half // _TM,),
        in_specs=in_specs,
        out_specs=[
            pl.BlockSpec((_TM, _C, _K), lambda i: (base + i, 0, 0)),
            pl.BlockSpec((_C, _TM), lambda i: (0, i)),
        ],
        out_shape=[
            jax.ShapeDtypeStruct((t, _C, _K), jnp.float32),
            jax.ShapeDtypeStruct((_C, half), jnp.int32),
        ],
        input_output_aliases=aliases,
    )(*args)


def _sc_gather(qc_seg, idx_row):
    n = idx_row.shape[1]
    mesh = plsc.VectorSubcoreMesh(core_axis_name="core",
                                  subcore_axis_name="subcore")

    @pl.kernel(out_type=jax.ShapeDtypeStruct((n, 128), jnp.float32),
               mesh=mesh)
    def gather_kernel(qc_hbm, i_hbm, o_hbm):
        def body(i_vmem, o_vmem):
            pltpu.sync_copy(qc_hbm.at[i_vmem.at[0]], o_vmem)

        pltpu.emit_pipeline(
            body,
            grid=(n // _GW,),
            in_specs=[pl.BlockSpec((1, _GW), lambda i: (0, i))],
            out_specs=[pl.BlockSpec((_GW, 128), lambda i: (i, 0))],
            core_axis_name=("core", "subcore"),
            dimension_semantics=(pltpu.PARALLEL,),
        )(i_hbm, o_hbm)

    return gather_kernel(qc_seg, idx_row)


def kernel(x, quantization_choices, W, b):
    B, S, _ = x.shape
    t = B * S
    xf = x.reshape(t, _DIN)
    wt = W.T.astype(jnp.bfloat16)
    wt0, wt1, b3d = wt[:, :_K], wt[:, _K:], b.reshape(1, _C, _K)
    logits, idx = _logits_and_indices(xf, wt0, wt1, b3d, 0, None)
    nseg = _D // 128
    a = idx.reshape(_C, t // 8, 8)
    seg = jax.lax.broadcasted_iota(jnp.int32, (_C, t // 8, nseg, 8), 2)
    idx6 = nseg * a[:, :, None, :] + seg
    idx6 = idx6.transpose(1, 0, 2, 3).reshape(1, t * _C * nseg)
    qc_seg = quantization_choices.reshape(_CK * (_D // 128), 128)
    rows = _sc_gather(qc_seg, idx6)
    q = (rows.reshape(t // 8, _C * (_D // 128), 8, 128)
         .transpose(0, 2, 1, 3).reshape(B, S, _C * _D))
    return q, logits.reshape(B, S, _C, _K)
